# baseline (speedup 1.0000x reference)
import functools

import jax
import jax.numpy as jnp
from jax import lax
from jax.experimental import pallas as pl
from jax.experimental.pallas import tpu as pltpu
from jax.experimental.pallas import tpu_sc as plsc

_H = 16
_L = 2048
_ROWS = 16
_BLOCKS = _L // _ROWS
_HALF_BLOCKS = _BLOCKS // 2


_DEPTH = 8


def _sc_fill_body(slabs_hbm, out_hbm, dslab_v, sem):
    c = lax.axis_index("c")
    s = lax.axis_index("s")
    h = s
    pltpu.sync_copy(slabs_hbm.at[h], dslab_v)

    def _copy(t):
        r0 = (c * _HALF_BLOCKS + t) * _ROWS
        start = _L - r0
        return pltpu.make_async_copy(dslab_v.at[:, pl.ds(start, _L)],
                                     out_hbm.at[h, pl.ds(r0, _ROWS), :], sem)

    def blk(t, carry):
        _copy(t).start()

        @pl.when(t >= _DEPTH)
        def _wait():
            _copy(t - _DEPTH).wait()
        return carry

    lax.fori_loop(0, _HALF_BLOCKS, blk, 0)

    def drain(t, carry):
        _copy(_HALF_BLOCKS - _DEPTH + t).wait()
        return carry

    lax.fori_loop(0, _DEPTH, drain, 0)


_sc_fill = functools.partial(
    pl.kernel,
    out_type=jax.ShapeDtypeStruct((_H, _L, _L), jnp.float32),
    scratch_types=[pltpu.VMEM((_ROWS, 2 * _L), jnp.float32),
                   pltpu.SemaphoreType.DMA],
    mesh=plsc.VectorSubcoreMesh(core_axis_name="c", subcore_axis_name="s"),
    compiler_params=pltpu.CompilerParams(use_tc_tiling_on_sc=False),
)(_sc_fill_body)


def kernel(rel_bias, L):
    del L
    dext = jnp.tile(rel_bias[:, ::-1], (1, 3))
    slabs = jnp.stack(
        [lax.slice_in_dim(dext, _L - 1 - a, _L - 1 - a + 2 * _L, axis=1)
         for a in range(_ROWS)], axis=1)
    return _sc_fill(slabs)


# speedup vs baseline: 2.7832x; 2.7832x over previous
import functools

import jax
import jax.numpy as jnp
from jax import lax
from jax.experimental import pallas as pl
from jax.experimental.pallas import tpu as pltpu
from jax.experimental.pallas import tpu_sc as plsc

_H = 16
_L = 2048
_ROWS = 16
_PHASES = 8
_BLOCKS = _L // 128
_UNITS_PER_WORKER = (_H * _PHASES) // 32
_DEPTH = 8


def _w2_body(row0_ref, w2_ref):
    row = row0_ref[0]
    x = jnp.broadcast_to(row, (128, _L))
    p = lax.broadcasted_iota(jnp.int32, (128, _L), 0)
    for t in range(7):
        sh = 1 << t
        x = jnp.where((p & sh) != 0, jnp.roll(x, sh, axis=1), x)
    w2_ref[0, :, :_L] = x
    w2_ref[0, :, _L:] = x


_w2 = pl.pallas_call(
    _w2_body,
    grid=(_H,),
    in_specs=[pl.BlockSpec((1, 1, _L), lambda h: (h, 0, 0))],
    out_specs=pl.BlockSpec((1, 128, 2 * _L), lambda h: (h, 0, 0)),
    out_shape=jax.ShapeDtypeStruct((_H, 128, 2 * _L), jnp.float32),
)


def _sc_fill_body(w2_hbm, out_hbm, slab_v, sem):
    c = lax.axis_index("c")
    s = lax.axis_index("s")
    w = s * 2 + c

    for k in range(_UNITS_PER_WORKER):
        u = w * _UNITS_PER_WORKER + k
        h = u // _PHASES
        b = u % _PHASES
        pltpu.sync_copy(
            w2_hbm.at[h, pl.ds(pl.multiple_of(_ROWS * b, _ROWS), _ROWS), :],
            slab_v)

        def _copy(t):
            r0 = pl.multiple_of(128 * t + _ROWS * b, _ROWS)
            m0 = pl.multiple_of(2 * _L - 128 * t, 128)
            return pltpu.make_async_copy(
                slab_v.at[:, pl.ds(m0, _L)],
                out_hbm.at[h, pl.ds(r0, _ROWS), :], sem)

        def blk(t, carry):
            _copy(t).start()

            @pl.when(t >= _DEPTH)
            def _wait():
                _copy(t - _DEPTH).wait()
            return carry

        lax.fori_loop(0, _BLOCKS, blk, 0)

        def drain(t, carry):
            _copy(_BLOCKS - _DEPTH + t).wait()
            return carry

        lax.fori_loop(0, _DEPTH, drain, 0)


_sc_fill = functools.partial(
    pl.kernel,
    out_type=jax.ShapeDtypeStruct((_H, _L, _L), jnp.float32),
    scratch_types=[pltpu.VMEM((_ROWS, 2 * _L), jnp.float32),
                   pltpu.SemaphoreType.DMA],
    mesh=plsc.VectorSubcoreMesh(core_axis_name="c", subcore_axis_name="s"),
    compiler_params=pltpu.CompilerParams(use_tc_tiling_on_sc=True),
)(_sc_fill_body)


def kernel(rel_bias, L):
    del L
    row0 = jnp.roll(rel_bias[:, ::-1], 1, axis=1)
    w2 = _w2(row0.reshape(_H, 1, _L))
    return _sc_fill(w2)


# speedup vs baseline: 2.8407x; 1.0206x over previous
import functools

import jax
import jax.numpy as jnp
from jax import lax
from jax.experimental import pallas as pl
from jax.experimental.pallas import tpu as pltpu
from jax.experimental.pallas import tpu_sc as plsc

_H = 16
_L = 2048
_ROWS = 16
_PHASES = 8
_BLOCKS = _L // 128
_UNITS_PER_WORKER = (_H * _PHASES) // 32
_DEPTH = 8
_W2W = 31 * 128


def _w2_body(row0_ref, w2_ref):
    row = row0_ref[0]
    x = jnp.broadcast_to(row, (128, _L))
    p = lax.broadcasted_iota(jnp.int32, (128, _L), 0)
    for t in range(7):
        sh = 1 << t
        x = jnp.where((p & sh) != 0, jnp.roll(x, sh, axis=1), x)
    w2_ref[0, :, :_L] = x
    w2_ref[0, :, _L:] = x[:, :_W2W - _L]


_w2 = pl.pallas_call(
    _w2_body,
    grid=(_H,),
    in_specs=[pl.BlockSpec((1, 1, _L), lambda h: (h, 0, 0))],
    out_specs=pl.BlockSpec((1, 128, _W2W), lambda h: (h, 0, 0)),
    out_shape=jax.ShapeDtypeStruct((_H, 128, _W2W), jnp.float32),
)


def _sc_fill_body(w2_hbm, out_hbm, slab_v, sem, stage_sem):
    c = lax.axis_index("c")
    s = lax.axis_index("s")
    w = s * 2 + c

    def _unit(k):
        u = w * _UNITS_PER_WORKER + k
        return u // _PHASES, u % _PHASES

    def _stage(k, kbuf):
        h, b = _unit(k)
        return pltpu.make_async_copy(
            w2_hbm.at[h, pl.ds(pl.multiple_of(_ROWS * b, _ROWS), _ROWS), :],
            slab_v.at[kbuf], stage_sem)

    _stage(0, 0).start()

    for k in range(_UNITS_PER_WORKER):
        h, b = _unit(k)
        buf = slab_v.at[k % 2]
        _stage(k, k % 2).wait()
        if k + 1 < _UNITS_PER_WORKER:
            _stage(k + 1, (k + 1) % 2).start()

        def _copy(t):
            r0 = pl.multiple_of(128 * t + _ROWS * b, _ROWS)
            m0 = pl.multiple_of((2 * _L - 128 * t) & (_L - 1), 128)
            return pltpu.make_async_copy(
                buf.at[:, pl.ds(m0, _L)],
                out_hbm.at[h, pl.ds(r0, _ROWS), :], sem)

        def blk(t, carry):
            _copy(t).start()

            @pl.when(t >= _DEPTH)
            def _wait():
                _copy(t - _DEPTH).wait()
            return carry

        lax.fori_loop(0, _BLOCKS, blk, 0)

        def drain(t, carry):
            _copy(_BLOCKS - _DEPTH + t).wait()
            return carry

        lax.fori_loop(0, _DEPTH, drain, 0)


_sc_fill = functools.partial(
    pl.kernel,
    out_type=jax.ShapeDtypeStruct((_H, _L, _L), jnp.float32),
    scratch_types=[pltpu.VMEM((2, _ROWS, _W2W), jnp.float32),
                   pltpu.SemaphoreType.DMA,
                   pltpu.SemaphoreType.DMA],
    mesh=plsc.VectorSubcoreMesh(core_axis_name="c", subcore_axis_name="s"),
    compiler_params=pltpu.CompilerParams(use_tc_tiling_on_sc=True),
)(_sc_fill_body)


def kernel(rel_bias, L):
    del L
    row0 = jnp.roll(rel_bias[:, ::-1], 1, axis=1)
    w2 = _w2(row0.reshape(_H, 1, _L))
    return _sc_fill(w2)


# speedup vs baseline: 3.0523x; 1.0745x over previous
import functools

import jax
import jax.numpy as jnp
from jax import lax
from jax.experimental import pallas as pl
from jax.experimental.pallas import tpu as pltpu
from jax.experimental.pallas import tpu_sc as plsc

_H = 16
_L = 2048
_ROWS = 16
_PHASES = 8
_BLOCKS = _L // 128
_UNITS_PER_WORKER = (_H * _PHASES) // 32
_DEPTH = 8
_W2W = 31 * 128


def _w2_body(row0_ref, w2_ref):
    row = row0_ref[0]
    x = jnp.broadcast_to(row, (8, _L))
    p = lax.broadcasted_iota(jnp.int32, (8, _L), 0)
    for t in range(3):
        sh = 1 << t
        x = jnp.where((p & sh) != 0, jnp.roll(x, sh, axis=1), x)
    n = 8
    while n < 128:
        x = jnp.concatenate([x, jnp.roll(x, n, axis=1)], axis=0)
        n *= 2
    w2_ref[0, :, :_L] = x
    w2_ref[0, :, _L:] = x[:, :_W2W - _L]


_w2 = pl.pallas_call(
    _w2_body,
    grid=(_H,),
    in_specs=[pl.BlockSpec((1, 1, _L), lambda h: (h, 0, 0))],
    out_specs=pl.BlockSpec((1, 128, _W2W), lambda h: (h, 0, 0)),
    out_shape=jax.ShapeDtypeStruct((_H, 128, _W2W), jnp.float32),
)


def _sc_fill_body(w2_hbm, out_hbm, slab_v, sem, stage_sem):
    c = lax.axis_index("c")
    s = lax.axis_index("s")
    w = s * 2 + c

    def _unit(k):
        u = w * _UNITS_PER_WORKER + k
        return u // _PHASES, u % _PHASES

    def _stage(k, kbuf):
        h, b = _unit(k)
        return pltpu.make_async_copy(
            w2_hbm.at[h, pl.ds(pl.multiple_of(_ROWS * b, _ROWS), _ROWS), :],
            slab_v.at[kbuf], stage_sem)

    _stage(0, 0).start()

    for k in range(_UNITS_PER_WORKER):
        h, b = _unit(k)
        buf = slab_v.at[k % 2]
        _stage(k, k % 2).wait()
        if k + 1 < _UNITS_PER_WORKER:
            _stage(k + 1, (k + 1) % 2).start()

        def _copy(t):
            r0 = pl.multiple_of(128 * t + _ROWS * b, _ROWS)
            m0 = pl.multiple_of((2 * _L - 128 * t) & (_L - 1), 128)
            return pltpu.make_async_copy(
                buf.at[:, pl.ds(m0, _L)],
                out_hbm.at[h, pl.ds(r0, _ROWS), :], sem)

        def blk(t, carry):
            _copy(t).start()

            @pl.when(t >= _DEPTH)
            def _wait():
                _copy(t - _DEPTH).wait()
            return carry

        lax.fori_loop(0, _BLOCKS, blk, 0)

        def drain(t, carry):
            _copy(_BLOCKS - _DEPTH + t).wait()
            return carry

        lax.fori_loop(0, _DEPTH, drain, 0)


_sc_fill = functools.partial(
    pl.kernel,
    out_type=jax.ShapeDtypeStruct((_H, _L, _L), jnp.float32),
    scratch_types=[pltpu.VMEM((2, _ROWS, _W2W), jnp.float32),
                   pltpu.SemaphoreType.DMA,
                   pltpu.SemaphoreType.DMA],
    mesh=plsc.VectorSubcoreMesh(core_axis_name="c", subcore_axis_name="s"),
    compiler_params=pltpu.CompilerParams(use_tc_tiling_on_sc=True),
)(_sc_fill_body)


def kernel(rel_bias, L):
    del L
    row0 = jnp.roll(rel_bias[:, ::-1], 1, axis=1)
    w2 = _w2(row0.reshape(_H, 1, _L))
    return _sc_fill(w2)


# speedup vs baseline: 3.1475x; 1.0312x over previous
import functools

import jax
import jax.numpy as jnp
from jax import lax
from jax.experimental import pallas as pl
from jax.experimental.pallas import tpu as pltpu
from jax.experimental.pallas import tpu_sc as plsc

_H = 16
_L = 2048
_ROWS = 16
_PHASES = 8
_BLOCKS = _L // 128
_UNITS_PER_WORKER = (_H * _PHASES) // 32
_DEPTH = 8


def _w_body(row0_ref, w_ref):
    row = row0_ref[0]
    x = jnp.broadcast_to(row, (8, _L))
    p = lax.broadcasted_iota(jnp.int32, (8, _L), 0)
    for t in range(3):
        sh = 1 << t
        x = jnp.where((p & sh) != 0, jnp.roll(x, sh, axis=1), x)
    n = 8
    while n < 128:
        x = jnp.concatenate([x, jnp.roll(x, n, axis=1)], axis=0)
        n *= 2
    w_ref[0] = x


_w = pl.pallas_call(
    _w_body,
    grid=(_H,),
    in_specs=[pl.BlockSpec((1, 1, _L), lambda h: (h, 0, 0))],
    out_specs=pl.BlockSpec((1, 128, _L), lambda h: (h, 0, 0)),
    out_shape=jax.ShapeDtypeStruct((_H, 128, _L), jnp.float32),
)


def _sc_fill_body(w_hbm, out_hbm, slab_v, sem, stage_sem):
    c = lax.axis_index("c")
    s = lax.axis_index("s")
    w = s * 2 + c

    def _unit(k):
        u = w * _UNITS_PER_WORKER + k
        return u // _PHASES, u % _PHASES

    def _stage(k, kbuf):
        h, b = _unit(k)
        return pltpu.make_async_copy(
            w_hbm.at[h, pl.ds(pl.multiple_of(_ROWS * b, _ROWS), _ROWS), :],
            slab_v.at[kbuf], stage_sem)

    _stage(0, 0).start()

    for k in range(_UNITS_PER_WORKER):
        h, b = _unit(k)
        buf = slab_v.at[k % 2]
        _stage(k, k % 2).wait()
        if k + 1 < _UNITS_PER_WORKER:
            _stage(k + 1, (k + 1) % 2).start()

        def _copies(B):
            r0 = pl.multiple_of(128 * B + _ROWS * b, _ROWS)
            n1 = _L - 128 * B
            out = [pltpu.make_async_copy(
                buf.at[:, pl.ds(0, n1)],
                out_hbm.at[h, pl.ds(r0, _ROWS), pl.ds(128 * B, n1)], sem)]
            if B > 0:
                out.append(pltpu.make_async_copy(
                    buf.at[:, pl.ds(n1, 128 * B)],
                    out_hbm.at[h, pl.ds(r0, _ROWS), pl.ds(0, 128 * B)], sem))
            return out

        for B in range(_BLOCKS):
            for d in _copies(B):
                d.start()
            if B >= _DEPTH:
                for d in _copies(B - _DEPTH):
                    d.wait()
        for B in range(_BLOCKS - _DEPTH, _BLOCKS):
            for d in _copies(B):
                d.wait()


_sc_fill = functools.partial(
    pl.kernel,
    out_type=jax.ShapeDtypeStruct((_H, _L, _L), jnp.float32),
    scratch_types=[pltpu.VMEM((2, _ROWS, _L), jnp.float32),
                   pltpu.SemaphoreType.DMA,
                   pltpu.SemaphoreType.DMA],
    mesh=plsc.VectorSubcoreMesh(core_axis_name="c", subcore_axis_name="s"),
    compiler_params=pltpu.CompilerParams(use_tc_tiling_on_sc=True),
)(_sc_fill_body)


def kernel(rel_bias, L):
    del L
    row0 = jnp.roll(rel_bias[:, ::-1], 1, axis=1)
    w = _w(row0.reshape(_H, 1, _L))
    return _sc_fill(w)


# speedup vs baseline: 3.1508x; 1.0010x over previous
import functools

import jax
import jax.numpy as jnp
from jax import lax
from jax.experimental import pallas as pl
from jax.experimental.pallas import tpu as pltpu
from jax.experimental.pallas import tpu_sc as plsc

_H = 16
_L = 2048
_ROWS = 16
_PHASES = 8
_BLOCKS = _L // 128
_UNITS_PER_WORKER = (_H * _PHASES) // 32
_DEPTH = 8


def _w_body(row0_ref, w_ref):
    row = row0_ref[0]
    x = jnp.broadcast_to(row, (8, _L))
    p = lax.broadcasted_iota(jnp.int32, (8, _L), 0)
    for t in range(3):
        sh = 1 << t
        x = jnp.where((p & sh) != 0, jnp.roll(x, sh, axis=1), x)
    n = 8
    while n < 64:
        x = jnp.concatenate([x, jnp.roll(x, n, axis=1)], axis=0)
        n *= 2
    w_ref[0, :64] = x
    w_ref[0, 64:] = jnp.roll(x, 64, axis=1)


_w = pl.pallas_call(
    _w_body,
    grid=(_H,),
    in_specs=[pl.BlockSpec((1, 1, _L), lambda h: (h, 0, 0))],
    out_specs=pl.BlockSpec((1, 128, _L), lambda h: (h, 0, 0)),
    out_shape=jax.ShapeDtypeStruct((_H, 128, _L), jnp.float32),
)


def _sc_fill_body(w_hbm, out_hbm, slab_v, sem, stage_sem):
    c = lax.axis_index("c")
    s = lax.axis_index("s")
    w = s * 2 + c

    def _unit(k):
        u = w * _UNITS_PER_WORKER + k
        return u // _PHASES, u % _PHASES

    def _stage(k, kbuf):
        h, b = _unit(k)
        return pltpu.make_async_copy(
            w_hbm.at[h, pl.ds(pl.multiple_of(_ROWS * b, _ROWS), _ROWS), :],
            slab_v.at[kbuf], stage_sem)

    _stage(0, 0).start()

    for k in range(_UNITS_PER_WORKER):
        h, b = _unit(k)
        buf = slab_v.at[k % 2]
        _stage(k, k % 2).wait()
        if k + 1 < _UNITS_PER_WORKER:
            _stage(k + 1, (k + 1) % 2).start()

        def _copies(B):
            r0 = pl.multiple_of(128 * B + _ROWS * b, _ROWS)
            n1 = _L - 128 * B
            out = [pltpu.make_async_copy(
                buf.at[:, pl.ds(0, n1)],
                out_hbm.at[h, pl.ds(r0, _ROWS), pl.ds(128 * B, n1)], sem)]
            if B > 0:
                out.append(pltpu.make_async_copy(
                    buf.at[:, pl.ds(n1, 128 * B)],
                    out_hbm.at[h, pl.ds(r0, _ROWS), pl.ds(0, 128 * B)], sem))
            return out

        for B in range(_BLOCKS):
            for d in _copies(B):
                d.start()
            if B >= _DEPTH:
                for d in _copies(B - _DEPTH):
                    d.wait()
        for B in range(_BLOCKS - _DEPTH, _BLOCKS):
            for d in _copies(B):
                d.wait()


_sc_fill = functools.partial(
    pl.kernel,
    out_type=jax.ShapeDtypeStruct((_H, _L, _L), jnp.float32),
    scratch_types=[pltpu.VMEM((2, _ROWS, _L), jnp.float32),
                   pltpu.SemaphoreType.DMA,
                   pltpu.SemaphoreType.DMA],
    mesh=plsc.VectorSubcoreMesh(core_axis_name="c", subcore_axis_name="s"),
    compiler_params=pltpu.CompilerParams(use_tc_tiling_on_sc=True),
)(_sc_fill_body)


def kernel(rel_bias, L):
    del L
    row0 = jnp.roll(rel_bias[:, ::-1], 1, axis=1)
    w = _w(row0.reshape(_H, 1, _L))
    return _sc_fill(w)


# speedup vs baseline: 3.1515x; 1.0002x over previous
import functools

import jax
import jax.numpy as jnp
from jax import lax
from jax.experimental import pallas as pl
from jax.experimental.pallas import tpu as pltpu
from jax.experimental.pallas import tpu_sc as plsc

_H = 16
_L = 2048
_ROWS = 16
_PHASES = 8
_BLOCKS = _L // 128
_UNITS_PER_WORKER = (_H * _PHASES) // 32
_DEPTH = 12


def _w_body(row0_ref, w_ref):
    row = row0_ref[0]
    x = jnp.broadcast_to(row, (8, _L))
    p = lax.broadcasted_iota(jnp.int32, (8, _L), 0)
    for t in range(3):
        sh = 1 << t
        x = jnp.where((p & sh) != 0, jnp.roll(x, sh, axis=1), x)
    n = 8
    while n < 64:
        x = jnp.concatenate([x, jnp.roll(x, n, axis=1)], axis=0)
        n *= 2
    w_ref[0, :64] = x
    w_ref[0, 64:] = jnp.roll(x, 64, axis=1)


_w = pl.pallas_call(
    _w_body,
    grid=(_H,),
    in_specs=[pl.BlockSpec((1, 1, _L), lambda h: (h, 0, 0))],
    out_specs=pl.BlockSpec((1, 128, _L), lambda h: (h, 0, 0)),
    out_shape=jax.ShapeDtypeStruct((_H, 128, _L), jnp.float32),
)


def _sc_fill_body(w_hbm, out_hbm, slab_v, sem, stage_sem):
    c = lax.axis_index("c")
    s = lax.axis_index("s")
    w = s * 2 + c

    def _unit(k):
        u = w * _UNITS_PER_WORKER + k
        return u // _PHASES, u % _PHASES

    def _stage(k, kbuf):
        h, b = _unit(k)
        return pltpu.make_async_copy(
            w_hbm.at[h, pl.ds(pl.multiple_of(_ROWS * b, _ROWS), _ROWS), :],
            slab_v.at[kbuf], stage_sem)

    _stage(0, 0).start()

    for k in range(_UNITS_PER_WORKER):
        h, b = _unit(k)
        buf = slab_v.at[k % 2]
        _stage(k, k % 2).wait()
        if k + 1 < _UNITS_PER_WORKER:
            _stage(k + 1, (k + 1) % 2).start()

        def _copies(B):
            r0 = pl.multiple_of(128 * B + _ROWS * b, _ROWS)
            n1 = _L - 128 * B
            out = [pltpu.make_async_copy(
                buf.at[:, pl.ds(0, n1)],
                out_hbm.at[h, pl.ds(r0, _ROWS), pl.ds(128 * B, n1)], sem)]
            if B > 0:
                out.append(pltpu.make_async_copy(
                    buf.at[:, pl.ds(n1, 128 * B)],
                    out_hbm.at[h, pl.ds(r0, _ROWS), pl.ds(0, 128 * B)], sem))
            return out

        for B in range(_BLOCKS):
            for d in _copies(B):
                d.start()
            if B >= _DEPTH:
                for d in _copies(B - _DEPTH):
                    d.wait()
        for B in range(_BLOCKS - _DEPTH, _BLOCKS):
            for d in _copies(B):
                d.wait()


_sc_fill = functools.partial(
    pl.kernel,
    out_type=jax.ShapeDtypeStruct((_H, _L, _L), jnp.float32),
    scratch_types=[pltpu.VMEM((2, _ROWS, _L), jnp.float32),
                   pltpu.SemaphoreType.DMA,
                   pltpu.SemaphoreType.DMA],
    mesh=plsc.VectorSubcoreMesh(core_axis_name="c", subcore_axis_name="s"),
    compiler_params=pltpu.CompilerParams(use_tc_tiling_on_sc=True),
)(_sc_fill_body)


def kernel(rel_bias, L):
    del L
    row0 = jnp.roll(rel_bias[:, ::-1], 1, axis=1)
    w = _w(row0.reshape(_H, 1, _L))
    return _sc_fill(w)


# speedup vs baseline: 3.1651x; 1.0043x over previous
import functools

import jax
import jax.numpy as jnp
from jax import lax
from jax.experimental import pallas as pl
from jax.experimental.pallas import tpu as pltpu
from jax.experimental.pallas import tpu_sc as plsc

_H = 16
_L = 2048
_ROWS = 16
_PHASES = 8
_BLOCKS = _L // 128
_UNITS_PER_WORKER = (_H * _PHASES) // 32
_DEPTH = 8


def _w_body(row0_ref, w_ref):
    row = row0_ref[0]
    x = jnp.broadcast_to(row, (8, _L))
    p = lax.broadcasted_iota(jnp.int32, (8, _L), 0)
    for t in range(3):
        sh = 1 << t
        x = jnp.where((p & sh) != 0, jnp.roll(x, sh, axis=1), x)
    n = 8
    while n < 64:
        x = jnp.concatenate([x, jnp.roll(x, n, axis=1)], axis=0)
        n *= 2
    w_ref[0, :64] = x
    w_ref[0, 64:] = jnp.roll(x, 64, axis=1)


_w = pl.pallas_call(
    _w_body,
    grid=(_H,),
    in_specs=[pl.BlockSpec((1, 1, _L), lambda h: (h, 0, 0))],
    out_specs=pl.BlockSpec((1, 128, _L), lambda h: (h, 0, 0)),
    out_shape=jax.ShapeDtypeStruct((_H, 128, _L), jnp.float32),
)


def _sc_fill_body(w_hbm, out_hbm, slab_v, sem, stage_sem):
    c = lax.axis_index("c")
    s = lax.axis_index("s")
    w = s * 2 + c

    def _unit(k):
        u = w * _UNITS_PER_WORKER + k
        return u // _PHASES, u % _PHASES

    def _stage(k, kbuf):
        h, b = _unit(k)
        return pltpu.make_async_copy(
            w_hbm.at[h, pl.ds(pl.multiple_of(_ROWS * b, _ROWS), _ROWS), :],
            slab_v.at[kbuf], stage_sem)

    _stage(0, 0).start()

    for k in range(_UNITS_PER_WORKER):
        h, b = _unit(k)
        buf = slab_v.at[k % 2]
        _stage(k, k % 2).wait()
        if k + 1 < _UNITS_PER_WORKER:
            _stage(k + 1, (k + 1) % 2).start()

        def _copies(B):
            r0 = pl.multiple_of(128 * B + _ROWS * b, _ROWS)
            n1 = _L - 128 * B
            out = [pltpu.make_async_copy(
                buf.at[:, pl.ds(0, n1)],
                out_hbm.at[h, pl.ds(r0, _ROWS), pl.ds(128 * B, n1)], sem)]
            if B > 0:
                out.append(pltpu.make_async_copy(
                    buf.at[:, pl.ds(n1, 128 * B)],
                    out_hbm.at[h, pl.ds(r0, _ROWS), pl.ds(0, 128 * B)], sem))
            return out

        for B in range(_BLOCKS):
            for d in _copies(B):
                d.start()
            if B >= _DEPTH:
                for d in _copies(B - _DEPTH):
                    d.wait()
        for B in range(_BLOCKS - _DEPTH, _BLOCKS):
            for d in _copies(B):
                d.wait()


_sc_fill = functools.partial(
    pl.kernel,
    out_type=jax.ShapeDtypeStruct((_H, _L, _L), jnp.float32),
    scratch_types=[pltpu.VMEM((2, _ROWS, _L), jnp.float32),
                   pltpu.SemaphoreType.DMA,
                   pltpu.SemaphoreType.DMA],
    mesh=plsc.VectorSubcoreMesh(core_axis_name="c", subcore_axis_name="s"),
    compiler_params=pltpu.CompilerParams(use_tc_tiling_on_sc=True),
)(_sc_fill_body)


def kernel(rel_bias, L):
    del L
    row0 = jnp.roll(rel_bias[:, ::-1], 1, axis=1)
    w = _w(row0.reshape(_H, 1, _L))
    return _sc_fill(w)
